# baseline (speedup 1.0000x reference)
import jax
import jax.numpy as jnp
from jax.experimental import pallas as pl
from jax.experimental.pallas import tpu as pltpu

_B, _P, _D = 64, 1024, 768
_TCB = 4


def _body(x_ref, t_ref, o_ref):
    o_ref[...] = x_ref[...] + t_ref[...]


def kernel(x, table):
    return pl.pallas_call(
        _body,
        grid=(_B // _TCB,),
        in_specs=[
            pl.BlockSpec((_TCB, _P, _D), lambda b: (b, 0, 0)),
            pl.BlockSpec((_P, _D), lambda b: (0, 0)),
        ],
        out_specs=pl.BlockSpec((_TCB, _P, _D), lambda b: (b, 0, 0)),
        out_shape=jax.ShapeDtypeStruct((_B, _P, _D), jnp.float32),
        compiler_params=pltpu.CompilerParams(
            dimension_semantics=("arbitrary",),
        ),
    )(x, table)
